# baseline (speedup 1.0000x reference)
import functools

import jax
import jax.numpy as jnp
from jax import lax
from jax.experimental import pallas as pl
from jax.experimental.pallas import tpu as pltpu
from jax.experimental.pallas import tpu_sc as plsc

N = 10000
D = 128
NCLS = 40
NITE = 4
SMOOTH = 0.5

NPAD = 10240
DHALF = NPAD // 2
PAD_IDX = N
NSUB = 16
CH = 128
KCH = 160
QCH = 16
QN = KCH // QCH
NB = 4
EPAD = NSUB * KCH * CH
RB = 1280
GRID = NPAD // RB
SROWS = DHALF // 16
IGN = -1


def _fix_local(idx_v, base):
    def fix(t, _):
        r = t // 8
        k = t % 8
        v = idx_v[r, pl.ds(k * 16, 16)]
        vl = v - base
        ok = jnp.logical_and(vl >= 0, vl < DHALF)
        idx_v[r, pl.ds(k * 16, 16)] = jnp.where(ok, vl, IGN)
        return 0
    lax.fori_loop(0, QCH * 8, fix, 0)


def _spmm_body(sup_hbm, src_hbm, dst_hbm, zD_hbm, agg_hbm,
               src_v, dst_v, buf0, buf1, buf2, buf3,
               sg0, sg1, sg2, sg3, ss0, ss1, ss2, ss3, agg_sh):
    cid = lax.axis_index("c")
    sid = lax.axis_index("s")
    base = cid * DHALF
    bufs = (buf0, buf1, buf2, buf3)
    sgs = (sg0, sg1, sg2, sg3)
    sss = (ss0, ss1, ss2, ss3)

    pltpu.sync_copy(zD_hbm.at[pl.ds(sid * SROWS, SROWS)],
                    agg_sh.at[pl.ds(sid * SROWS, SROWS)])
    plsc.subcore_barrier()

    def _gather(row, buf, sem):
        pltpu.async_copy(sup_hbm.at[src_v.at[row]], buf, sem)

    def _gather_wait(row, buf, sem):
        pltpu.make_async_copy(sup_hbm.at[src_v.at[row]], buf, sem).wait()

    def _dref(row):
        return agg_sh.at[plsc.Indices(dst_v.at[row], ignored_value=IGN)]

    def _scatter(row, buf, sem):
        pltpu.async_copy(buf, _dref(row), sem, add=True)

    def _scatter_wait(row, buf, sem):
        pltpu.make_async_copy(buf, _dref(row), sem).wait()

    def group(g, _):
        pltpu.sync_copy(src_hbm.at[sid, pl.ds(g * QCH, QCH)], src_v)
        pltpu.sync_copy(dst_hbm.at[sid, pl.ds(g * QCH, QCH)], dst_v)
        _fix_local(dst_v, base)
        for k in range(NB):
            _gather(k, bufs[k], sgs[k])

        def rnd(j, _):
            for k in range(NB):
                c = NB * j + k
                _gather_wait(c, bufs[k], sgs[k])
                _scatter(c, bufs[k], sss[k])
            for k in range(NB):
                c = NB * j + k
                _scatter_wait(c, bufs[k], sss[k])
                _gather(c + NB, bufs[k], sgs[k])
            return 0
        lax.fori_loop(0, QCH // NB - 1, rnd, 0)

        tb = QCH - NB
        for k in range(NB):
            _gather_wait(tb + k, bufs[k], sgs[k])
            _scatter(tb + k, bufs[k], sss[k])
        for k in range(NB):
            _scatter_wait(tb + k, bufs[k], sss[k])
        return 0
    lax.fori_loop(0, QN, group, 0)
    plsc.subcore_barrier()

    pltpu.sync_copy(agg_sh.at[pl.ds(sid * SROWS, SROWS)],
                    agg_hbm.at[cid, pl.ds(sid * SROWS, SROWS)])


def _deg_body(src_hbm, dst_hbm, zD_hbm, olo_hbm, ohi_hbm, deg_hbm,
              src_v, dst_v, olo_v, ohi_v, acc_sh, sem2, sem3):
    cid = lax.axis_index("c")
    sid = lax.axis_index("s")
    base = cid * DHALF

    pltpu.sync_copy(zD_hbm.at[pl.ds(sid * SROWS, SROWS)],
                    acc_sh.at[pl.ds(sid * SROWS, SROWS)])
    pltpu.sync_copy(olo_hbm, olo_v)
    pltpu.sync_copy(ohi_hbm, ohi_v)
    plsc.subcore_barrier()

    def _sref(idx_v, row):
        return acc_sh.at[plsc.Indices(idx_v.at[row], ignored_value=IGN)]

    def group(g, _):
        pltpu.sync_copy(src_hbm.at[sid, pl.ds(g * QCH, QCH)], src_v)
        pltpu.sync_copy(dst_hbm.at[sid, pl.ds(g * QCH, QCH)], dst_v)
        _fix_local(src_v, base)
        _fix_local(dst_v, base)

        def fire(j, _):
            pltpu.async_copy(olo_v, _sref(src_v, j), sem2, add=True)
            pltpu.async_copy(ohi_v, _sref(dst_v, j), sem3, add=True)
            return 0
        lax.fori_loop(0, QCH, fire, 0)

        def drain(j, _):
            pltpu.make_async_copy(olo_v, _sref(src_v, j), sem2).wait()
            pltpu.make_async_copy(ohi_v, _sref(dst_v, j), sem3).wait()
            return 0
        lax.fori_loop(0, QCH, drain, 0)
        return 0
    lax.fori_loop(0, QN, group, 0)
    plsc.subcore_barrier()

    pltpu.sync_copy(acc_sh.at[pl.ds(sid * SROWS, SROWS)],
                    deg_hbm.at[cid, pl.ds(sid * SROWS, SROWS)])


@functools.lru_cache(maxsize=1)
def _sc_kernels():
    mesh = plsc.VectorSubcoreMesh(core_axis_name="c", subcore_axis_name="s",
                                  num_cores=2, num_subcores=16)
    spmm = pl.kernel(
        _spmm_body,
        out_type=jax.ShapeDtypeStruct((2, DHALF, D), jnp.float32),
        mesh=mesh,
        scratch_types=[
            pltpu.VMEM((QCH, CH), jnp.int32),
            pltpu.VMEM((QCH, CH), jnp.int32),
            pltpu.VMEM((CH, D), jnp.float32),
            pltpu.VMEM((CH, D), jnp.float32),
            pltpu.VMEM((CH, D), jnp.float32),
            pltpu.VMEM((CH, D), jnp.float32),
            pltpu.SemaphoreType.DMA,
            pltpu.SemaphoreType.DMA,
            pltpu.SemaphoreType.DMA,
            pltpu.SemaphoreType.DMA,
            pltpu.SemaphoreType.DMA,
            pltpu.SemaphoreType.DMA,
            pltpu.SemaphoreType.DMA,
            pltpu.SemaphoreType.DMA,
            pltpu.VMEM_SHARED((DHALF, D), jnp.float32),
        ],
    )
    deg = pl.kernel(
        _deg_body,
        out_type=jax.ShapeDtypeStruct((2, DHALF, D), jnp.float32),
        mesh=mesh,
        scratch_types=[
            pltpu.VMEM((QCH, CH), jnp.int32),
            pltpu.VMEM((QCH, CH), jnp.int32),
            pltpu.VMEM((CH, D), jnp.float32),
            pltpu.VMEM((CH, D), jnp.float32),
            pltpu.VMEM_SHARED((DHALF, D), jnp.float32),
            pltpu.SemaphoreType.DMA,
            pltpu.SemaphoreType.DMA,
        ],
    )
    return spmm, deg


def _col(dref, lane):
    s = dref[:, lane:lane + 1]
    return lax.rsqrt(jnp.maximum(s, 1.0))


def _kpre_body(x_ref, w_ref, deg_ref, sup_ref):
    f = _col(deg_ref, 0)
    sup_ref[...] = jnp.dot(x_ref[...], w_ref[...],
                           preferred_element_type=jnp.float32) * f


def _kmid_body(h_ref, agg_ref, deg_ref, w_ref, b_ref, h_out, sup_out):
    f = _col(deg_ref, 0)
    g = _col(deg_ref, 64)
    aggn = g * agg_ref[...] + b_ref[...]
    hn = jnp.maximum(SMOOTH * h_ref[...] + (1.0 - SMOOTH) * aggn, 0.0)
    h_out[...] = hn
    sup_out[...] = jnp.dot(hn, w_ref[...],
                           preferred_element_type=jnp.float32) * f


def _kpost_body(h_ref, agg_ref, deg_ref, b_ref, wl_ref, out_ref):
    g = _col(deg_ref, 64)
    aggn = g * agg_ref[...] + b_ref[...]
    hn = jnp.maximum(SMOOTH * h_ref[...] + (1.0 - SMOOTH) * aggn, 0.0)
    logits = jnp.dot(hn, wl_ref[...], preferred_element_type=jnp.float32)
    m = jnp.max(logits, axis=1, keepdims=True)
    lse = m + jnp.log(jnp.sum(jnp.exp(logits - m), axis=1, keepdims=True))
    out_ref[...] = logits - lse


def _row_spec():
    return pl.BlockSpec((RB, D), lambda i: (i, 0))


def kernel(x, edge_index, W_gc, b_gc, W_lin):
    src = edge_index[0]
    dst = edge_index[1]
    npad_e = EPAD - src.shape[0]
    pad = jnp.full((npad_e,), PAD_IDX, jnp.int32)
    src_p = jnp.concatenate([src, pad]).reshape(NSUB, KCH, CH)
    dst_p = jnp.concatenate([dst, pad]).reshape(NSUB, KCH, CH)
    x_p = jnp.pad(x, ((0, NPAD - N), (0, 0)))
    b2 = b_gc.reshape(1, D)
    zD = jnp.zeros((NPAD, D), jnp.float32)
    lanes = jnp.arange(D)
    olo = jnp.broadcast_to((lanes < 64).astype(jnp.float32), (CH, D))
    ohi = jnp.broadcast_to((lanes >= 64).astype(jnp.float32), (CH, D))

    spmm_kernel, deg_kernel = _sc_kernels()
    degs = deg_kernel(src_p, dst_p, zD, olo, ohi).reshape(NPAD, D)

    kpre = pl.pallas_call(
        _kpre_body,
        grid=(GRID,),
        in_specs=[
            _row_spec(),
            pl.BlockSpec((D, D), lambda i: (0, 0)),
            _row_spec(),
        ],
        out_specs=_row_spec(),
        out_shape=jax.ShapeDtypeStruct((NPAD, D), jnp.float32),
    )
    sup = kpre(x_p, W_gc, degs)

    kmid = pl.pallas_call(
        _kmid_body,
        grid=(GRID,),
        in_specs=[
            _row_spec(),
            _row_spec(),
            _row_spec(),
            pl.BlockSpec((D, D), lambda i: (0, 0)),
            pl.BlockSpec((1, D), lambda i: (0, 0)),
        ],
        out_specs=[_row_spec(), _row_spec()],
        out_shape=[jax.ShapeDtypeStruct((NPAD, D), jnp.float32)] * 2,
    )

    h = x_p
    for _ in range(NITE - 1):
        agg = spmm_kernel(sup, src_p, dst_p, zD).reshape(NPAD, D)
        h, sup = kmid(h, agg, degs, W_gc, b2)

    agg = spmm_kernel(sup, src_p, dst_p, zD).reshape(NPAD, D)

    kpost = pl.pallas_call(
        _kpost_body,
        grid=(GRID,),
        in_specs=[
            _row_spec(),
            _row_spec(),
            _row_spec(),
            pl.BlockSpec((1, D), lambda i: (0, 0)),
            pl.BlockSpec((D, NCLS), lambda i: (0, 0)),
        ],
        out_specs=pl.BlockSpec((RB, NCLS), lambda i: (i, 0)),
        out_shape=jax.ShapeDtypeStruct((NPAD, NCLS), jnp.float32),
    )
    out = kpost(h, agg, degs, b2, W_lin)
    return out[:N]


# speedup vs baseline: 1.2551x; 1.2551x over previous
import functools

import jax
import jax.numpy as jnp
from jax import lax
from jax.experimental import pallas as pl
from jax.experimental.pallas import tpu as pltpu
from jax.experimental.pallas import tpu_sc as plsc

N = 10000
D = 128
NCLS = 40
NITE = 4
SMOOTH = 0.5

NPAD = 10240
DHALF = NPAD // 2
PAD_IDX = N
NSUB = 16
CH = 128
KCH = 160
QCH = 16
QN = KCH // QCH
NB = 4
EPAD = NSUB * KCH * CH
RB = 1280
GRID = NPAD // RB
SROWS = DHALF // 16
IGN = -1


def _fix_local(idx_v, base):
    def fix(t, _):
        r = t // 8
        k = t % 8
        v = idx_v[r, pl.ds(k * 16, 16)]
        vl = v - base
        ok = jnp.logical_and(vl >= 0, vl < DHALF)
        idx_v[r, pl.ds(k * 16, 16)] = jnp.where(ok, vl, IGN)
        return 0
    lax.fori_loop(0, QCH * 8, fix, 0)


def _fix_pair(src_v, dst_v, base):
    def fix(t, _):
        r = t // 8
        k = t % 8
        s = src_v[r, pl.ds(k * 16, 16)]
        d = dst_v[r, pl.ds(k * 16, 16)]
        dl = d - base
        ok = jnp.logical_and(dl >= 0, dl < DHALF)
        dst_v[r, pl.ds(k * 16, 16)] = jnp.where(ok, dl, IGN)
        src_v[r, pl.ds(k * 16, 16)] = jnp.where(ok, s, IGN)
        return 0
    lax.fori_loop(0, QCH * 8, fix, 0)


def _spmm_body(sup_hbm, src_hbm, dst_hbm, zD_hbm, agg_hbm,
               src_v, dst_v, buf0, buf1, buf2, buf3,
               sg0, sg1, sg2, sg3, ss0, ss1, ss2, ss3, agg_sh):
    cid = lax.axis_index("c")
    sid = lax.axis_index("s")
    base = cid * DHALF
    bufs = (buf0, buf1, buf2, buf3)
    sgs = (sg0, sg1, sg2, sg3)
    sss = (ss0, ss1, ss2, ss3)

    pltpu.sync_copy(zD_hbm.at[pl.ds(sid * SROWS, SROWS)],
                    agg_sh.at[pl.ds(sid * SROWS, SROWS)])
    plsc.subcore_barrier()

    def _sref(row):
        return sup_hbm.at[plsc.Indices(src_v.at[row], ignored_value=IGN)]

    def _gather(row, buf, sem):
        pltpu.async_copy(_sref(row), buf, sem)

    def _gather_wait(row, buf, sem):
        pltpu.make_async_copy(_sref(row), buf, sem).wait()

    def _dref(row):
        return agg_sh.at[plsc.Indices(dst_v.at[row], ignored_value=IGN)]

    def _scatter(row, buf, sem):
        pltpu.async_copy(buf, _dref(row), sem, add=True)

    def _scatter_wait(row, buf, sem):
        pltpu.make_async_copy(buf, _dref(row), sem).wait()

    def group(g, _):
        pltpu.sync_copy(src_hbm.at[sid, pl.ds(g * QCH, QCH)], src_v)
        pltpu.sync_copy(dst_hbm.at[sid, pl.ds(g * QCH, QCH)], dst_v)
        _fix_pair(src_v, dst_v, base)
        for k in range(NB):
            _gather(k, bufs[k], sgs[k])

        def rnd(j, _):
            for k in range(NB):
                c = NB * j + k
                _gather_wait(c, bufs[k], sgs[k])
                _scatter(c, bufs[k], sss[k])
            for k in range(NB):
                c = NB * j + k
                _scatter_wait(c, bufs[k], sss[k])
                _gather(c + NB, bufs[k], sgs[k])
            return 0
        lax.fori_loop(0, QCH // NB - 1, rnd, 0)

        tb = QCH - NB
        for k in range(NB):
            _gather_wait(tb + k, bufs[k], sgs[k])
            _scatter(tb + k, bufs[k], sss[k])
        for k in range(NB):
            _scatter_wait(tb + k, bufs[k], sss[k])
        return 0
    lax.fori_loop(0, QN, group, 0)
    plsc.subcore_barrier()

    pltpu.sync_copy(agg_sh.at[pl.ds(sid * SROWS, SROWS)],
                    agg_hbm.at[cid, pl.ds(sid * SROWS, SROWS)])


def _deg_body(src_hbm, dst_hbm, zD_hbm, olo_hbm, ohi_hbm, deg_hbm,
              src_v, dst_v, olo_v, ohi_v, acc_sh, sem2, sem3):
    cid = lax.axis_index("c")
    sid = lax.axis_index("s")
    base = cid * DHALF

    pltpu.sync_copy(zD_hbm.at[pl.ds(sid * SROWS, SROWS)],
                    acc_sh.at[pl.ds(sid * SROWS, SROWS)])
    pltpu.sync_copy(olo_hbm, olo_v)
    pltpu.sync_copy(ohi_hbm, ohi_v)
    plsc.subcore_barrier()

    def _sref(idx_v, row):
        return acc_sh.at[plsc.Indices(idx_v.at[row], ignored_value=IGN)]

    def group(g, _):
        pltpu.sync_copy(src_hbm.at[sid, pl.ds(g * QCH, QCH)], src_v)
        pltpu.sync_copy(dst_hbm.at[sid, pl.ds(g * QCH, QCH)], dst_v)
        _fix_local(src_v, base)
        _fix_local(dst_v, base)

        def fire(j, _):
            pltpu.async_copy(olo_v, _sref(src_v, j), sem2, add=True)
            pltpu.async_copy(ohi_v, _sref(dst_v, j), sem3, add=True)
            return 0
        lax.fori_loop(0, QCH, fire, 0)

        def drain(j, _):
            pltpu.make_async_copy(olo_v, _sref(src_v, j), sem2).wait()
            pltpu.make_async_copy(ohi_v, _sref(dst_v, j), sem3).wait()
            return 0
        lax.fori_loop(0, QCH, drain, 0)
        return 0
    lax.fori_loop(0, QN, group, 0)
    plsc.subcore_barrier()

    pltpu.sync_copy(acc_sh.at[pl.ds(sid * SROWS, SROWS)],
                    deg_hbm.at[cid, pl.ds(sid * SROWS, SROWS)])


@functools.lru_cache(maxsize=1)
def _sc_kernels():
    mesh = plsc.VectorSubcoreMesh(core_axis_name="c", subcore_axis_name="s",
                                  num_cores=2, num_subcores=16)
    spmm = pl.kernel(
        _spmm_body,
        out_type=jax.ShapeDtypeStruct((2, DHALF, D), jnp.float32),
        mesh=mesh,
        scratch_types=[
            pltpu.VMEM((QCH, CH), jnp.int32),
            pltpu.VMEM((QCH, CH), jnp.int32),
            pltpu.VMEM((CH, D), jnp.float32),
            pltpu.VMEM((CH, D), jnp.float32),
            pltpu.VMEM((CH, D), jnp.float32),
            pltpu.VMEM((CH, D), jnp.float32),
            pltpu.SemaphoreType.DMA,
            pltpu.SemaphoreType.DMA,
            pltpu.SemaphoreType.DMA,
            pltpu.SemaphoreType.DMA,
            pltpu.SemaphoreType.DMA,
            pltpu.SemaphoreType.DMA,
            pltpu.SemaphoreType.DMA,
            pltpu.SemaphoreType.DMA,
            pltpu.VMEM_SHARED((DHALF, D), jnp.float32),
        ],
    )
    deg = pl.kernel(
        _deg_body,
        out_type=jax.ShapeDtypeStruct((2, DHALF, D), jnp.float32),
        mesh=mesh,
        scratch_types=[
            pltpu.VMEM((QCH, CH), jnp.int32),
            pltpu.VMEM((QCH, CH), jnp.int32),
            pltpu.VMEM((CH, D), jnp.float32),
            pltpu.VMEM((CH, D), jnp.float32),
            pltpu.VMEM_SHARED((DHALF, D), jnp.float32),
            pltpu.SemaphoreType.DMA,
            pltpu.SemaphoreType.DMA,
        ],
    )
    return spmm, deg


def _col(dref, lane):
    s = dref[:, lane:lane + 1]
    return lax.rsqrt(jnp.maximum(s, 1.0))


def _kpre_body(x_ref, w_ref, deg_ref, sup_ref):
    f = _col(deg_ref, 0)
    sup_ref[...] = jnp.dot(x_ref[...], w_ref[...],
                           preferred_element_type=jnp.float32) * f


def _kmid_body(h_ref, agg_ref, deg_ref, w_ref, b_ref, h_out, sup_out):
    f = _col(deg_ref, 0)
    g = _col(deg_ref, 64)
    aggn = g * agg_ref[...] + b_ref[...]
    hn = jnp.maximum(SMOOTH * h_ref[...] + (1.0 - SMOOTH) * aggn, 0.0)
    h_out[...] = hn
    sup_out[...] = jnp.dot(hn, w_ref[...],
                           preferred_element_type=jnp.float32) * f


def _kpost_body(h_ref, agg_ref, deg_ref, b_ref, wl_ref, out_ref):
    g = _col(deg_ref, 64)
    aggn = g * agg_ref[...] + b_ref[...]
    hn = jnp.maximum(SMOOTH * h_ref[...] + (1.0 - SMOOTH) * aggn, 0.0)
    logits = jnp.dot(hn, wl_ref[...], preferred_element_type=jnp.float32)
    m = jnp.max(logits, axis=1, keepdims=True)
    lse = m + jnp.log(jnp.sum(jnp.exp(logits - m), axis=1, keepdims=True))
    out_ref[...] = logits - lse


def _row_spec():
    return pl.BlockSpec((RB, D), lambda i: (i, 0))


def kernel(x, edge_index, W_gc, b_gc, W_lin):
    src = edge_index[0]
    dst = edge_index[1]
    npad_e = EPAD - src.shape[0]
    pad = jnp.full((npad_e,), PAD_IDX, jnp.int32)
    src_p = jnp.concatenate([src, pad]).reshape(NSUB, KCH, CH)
    dst_p = jnp.concatenate([dst, pad]).reshape(NSUB, KCH, CH)
    x_p = jnp.pad(x, ((0, NPAD - N), (0, 0)))
    b2 = b_gc.reshape(1, D)
    zD = jnp.zeros((NPAD, D), jnp.float32)
    lanes = jnp.arange(D)
    olo = jnp.broadcast_to((lanes < 64).astype(jnp.float32), (CH, D))
    ohi = jnp.broadcast_to((lanes >= 64).astype(jnp.float32), (CH, D))

    spmm_kernel, deg_kernel = _sc_kernels()
    degs = deg_kernel(src_p, dst_p, zD, olo, ohi).reshape(NPAD, D)

    kpre = pl.pallas_call(
        _kpre_body,
        grid=(GRID,),
        in_specs=[
            _row_spec(),
            pl.BlockSpec((D, D), lambda i: (0, 0)),
            _row_spec(),
        ],
        out_specs=_row_spec(),
        out_shape=jax.ShapeDtypeStruct((NPAD, D), jnp.float32),
    )
    sup = kpre(x_p, W_gc, degs)

    kmid = pl.pallas_call(
        _kmid_body,
        grid=(GRID,),
        in_specs=[
            _row_spec(),
            _row_spec(),
            _row_spec(),
            pl.BlockSpec((D, D), lambda i: (0, 0)),
            pl.BlockSpec((1, D), lambda i: (0, 0)),
        ],
        out_specs=[_row_spec(), _row_spec()],
        out_shape=[jax.ShapeDtypeStruct((NPAD, D), jnp.float32)] * 2,
    )

    h = x_p
    for _ in range(NITE - 1):
        agg = spmm_kernel(sup, src_p, dst_p, zD).reshape(NPAD, D)
        h, sup = kmid(h, agg, degs, W_gc, b2)

    agg = spmm_kernel(sup, src_p, dst_p, zD).reshape(NPAD, D)

    kpost = pl.pallas_call(
        _kpost_body,
        grid=(GRID,),
        in_specs=[
            _row_spec(),
            _row_spec(),
            _row_spec(),
            pl.BlockSpec((1, D), lambda i: (0, 0)),
            pl.BlockSpec((D, NCLS), lambda i: (0, 0)),
        ],
        out_specs=pl.BlockSpec((RB, NCLS), lambda i: (i, 0)),
        out_shape=jax.ShapeDtypeStruct((NPAD, NCLS), jnp.float32),
    )
    out = kpost(h, agg, degs, b2, W_lin)
    return out[:N]


# speedup vs baseline: 1.4810x; 1.1800x over previous
import functools

import jax
import jax.numpy as jnp
from jax import lax
from jax.experimental import pallas as pl
from jax.experimental.pallas import tpu as pltpu
from jax.experimental.pallas import tpu_sc as plsc

N = 10000
D = 128
NCLS = 40
NITE = 4
SMOOTH = 0.5

NPAD = 10240
SHALF = NPAD // 2
PAD_IDX = N
NW = 32
CH = 128
KCH = 80
QN = 5
QCH = KCH // QN
NB = 4
EPAD = NW * KCH * CH
RB = 1280
GRID = NPAD // RB
SROWS = NPAD // 16
HROWS = SHALF // 16
IGN = -1


def _wid():
    return lax.axis_index("s") * 2 + lax.axis_index("c")


def _spmm_body(sup_hbm, src_hbm, dst_hbm, zD_hbm, agg_hbm,
               src_v, dst_v, buf0, buf1, buf2, buf3, agg_sh,
               sem0, sem1, sem2, sem3):
    cid = lax.axis_index("c")
    sid = lax.axis_index("s")
    wid = _wid()
    bufs = (buf0, buf1, buf2, buf3)
    sems = (sem0, sem1, sem2, sem3)

    def _sref(row):
        return sup_hbm.at[plsc.Indices(src_v.at[row], ignored_value=IGN)]

    def _dref(row):
        return agg_sh.at[plsc.Indices(dst_v.at[row], ignored_value=IGN)]

    for k in range(2):
        pltpu.sync_copy(zD_hbm.at[pl.ds(sid * HROWS, HROWS)],
                        agg_sh.at[pl.ds(sid * HROWS, HROWS)])
        plsc.subcore_barrier()

        def quarter(q, _):
            pltpu.sync_copy(src_hbm.at[k, wid, pl.ds(q * QCH, QCH)], src_v)
            pltpu.sync_copy(dst_hbm.at[k, wid, pl.ds(q * QCH, QCH)], dst_v)
            for t in range(NB):
                pltpu.async_copy(_sref(t), bufs[t], sems[t])

            def rnd(j, _):
                for t in range(NB):
                    c = NB * j + t
                    pltpu.make_async_copy(_sref(c), bufs[t], sems[t]).wait()
                    pltpu.sync_copy(bufs[t], _dref(c), add=True)
                    pltpu.async_copy(_sref(c + NB), bufs[t], sems[t])
                return 0
            lax.fori_loop(0, QCH // NB - 1, rnd, 0)

            tb = QCH - NB
            for t in range(NB):
                pltpu.make_async_copy(_sref(tb + t), bufs[t], sems[t]).wait()
                pltpu.sync_copy(bufs[t], _dref(tb + t), add=True)
            return 0
        lax.fori_loop(0, QN, quarter, 0)
        plsc.subcore_barrier()

        pltpu.sync_copy(
            agg_sh.at[pl.ds(sid * HROWS, HROWS)],
            agg_hbm.at[cid, pl.ds(k * SHALF + sid * HROWS, HROWS)])
        plsc.subcore_barrier()


def _deg_body(src_hbm, dst_hbm, zD_hbm, olo_hbm, ohi_hbm, deg_hbm,
              src_v, dst_v, olo_v, ohi_v, acc_sh, sem2, sem3):
    cid = lax.axis_index("c")
    sid = lax.axis_index("s")
    wid = sid * 2 + cid

    pltpu.sync_copy(zD_hbm.at[pl.ds(sid * SROWS, SROWS)],
                    acc_sh.at[pl.ds(sid * SROWS, SROWS)])
    pltpu.sync_copy(olo_hbm, olo_v)
    pltpu.sync_copy(ohi_hbm, ohi_v)
    plsc.subcore_barrier()

    def quarter(q, _):
        pltpu.sync_copy(src_hbm.at[wid, pl.ds(q * QCH, QCH)], src_v)
        pltpu.sync_copy(dst_hbm.at[wid, pl.ds(q * QCH, QCH)], dst_v)

        def fire(j, _):
            pltpu.async_copy(olo_v, acc_sh.at[src_v.at[j]], sem2, add=True)
            pltpu.async_copy(ohi_v, acc_sh.at[dst_v.at[j]], sem3, add=True)
            return 0
        lax.fori_loop(0, QCH, fire, 0)

        def drain(j, _):
            pltpu.make_async_copy(olo_v, acc_sh.at[src_v.at[j]], sem2).wait()
            pltpu.make_async_copy(ohi_v, acc_sh.at[dst_v.at[j]], sem3).wait()
            return 0
        lax.fori_loop(0, QCH, drain, 0)
        return 0
    lax.fori_loop(0, QN, quarter, 0)
    plsc.subcore_barrier()

    pltpu.sync_copy(acc_sh.at[pl.ds(sid * SROWS, SROWS)],
                    deg_hbm.at[cid, pl.ds(sid * SROWS, SROWS)])


@functools.lru_cache(maxsize=1)
def _sc_kernels():
    mesh = plsc.VectorSubcoreMesh(core_axis_name="c", subcore_axis_name="s",
                                  num_cores=2, num_subcores=16)
    spmm = pl.kernel(
        _spmm_body,
        out_type=jax.ShapeDtypeStruct((2, NPAD, D), jnp.float32),
        mesh=mesh,
        scratch_types=[
            pltpu.VMEM((QCH, CH), jnp.int32),
            pltpu.VMEM((QCH, CH), jnp.int32),
            pltpu.VMEM((CH, D), jnp.float32),
            pltpu.VMEM((CH, D), jnp.float32),
            pltpu.VMEM((CH, D), jnp.float32),
            pltpu.VMEM((CH, D), jnp.float32),
            pltpu.VMEM_SHARED((SHALF, D), jnp.float32),
            pltpu.SemaphoreType.DMA,
            pltpu.SemaphoreType.DMA,
            pltpu.SemaphoreType.DMA,
            pltpu.SemaphoreType.DMA,
        ],
    )
    deg = pl.kernel(
        _deg_body,
        out_type=jax.ShapeDtypeStruct((2, NPAD, D), jnp.float32),
        mesh=mesh,
        scratch_types=[
            pltpu.VMEM((QCH, CH), jnp.int32),
            pltpu.VMEM((QCH, CH), jnp.int32),
            pltpu.VMEM((CH, D), jnp.float32),
            pltpu.VMEM((CH, D), jnp.float32),
            pltpu.VMEM_SHARED((NPAD, D), jnp.float32),
            pltpu.SemaphoreType.DMA,
            pltpu.SemaphoreType.DMA,
        ],
    )
    return spmm, deg


def _col(dref, lane):
    s = dref[0, :, lane:lane + 1] + dref[1, :, lane:lane + 1]
    return lax.rsqrt(jnp.maximum(s, 1.0))


def _kpre_body(x_ref, w_ref, deg_ref, sup_ref):
    f = _col(deg_ref, 0)
    sup_ref[...] = jnp.dot(x_ref[...], w_ref[...],
                           preferred_element_type=jnp.float32) * f


def _kmid_body(h_ref, agg_ref, deg_ref, w_ref, b_ref,
               h_out, sup_out):
    f = _col(deg_ref, 0)
    g = _col(deg_ref, 64)
    agg = agg_ref[0] + agg_ref[1]
    aggn = g * agg + b_ref[...]
    hn = jnp.maximum(SMOOTH * h_ref[...] + (1.0 - SMOOTH) * aggn, 0.0)
    h_out[...] = hn
    sup_out[...] = jnp.dot(hn, w_ref[...],
                           preferred_element_type=jnp.float32) * f


def _kpost_body(h_ref, agg_ref, deg_ref, b_ref, wl_ref, out_ref):
    g = _col(deg_ref, 64)
    agg = agg_ref[0] + agg_ref[1]
    aggn = g * agg + b_ref[...]
    hn = jnp.maximum(SMOOTH * h_ref[...] + (1.0 - SMOOTH) * aggn, 0.0)
    logits = jnp.dot(hn, wl_ref[...], preferred_element_type=jnp.float32)
    m = jnp.max(logits, axis=1, keepdims=True)
    lse = m + jnp.log(jnp.sum(jnp.exp(logits - m), axis=1, keepdims=True))
    out_ref[...] = logits - lse


def _row_spec():
    return pl.BlockSpec((RB, D), lambda i: (i, 0))


def _agg_spec():
    return pl.BlockSpec((2, RB, D), lambda i: (0, i, 0))


def kernel(x, edge_index, W_gc, b_gc, W_lin):
    src = edge_index[0]
    dst = edge_index[1]
    npad_e = EPAD - src.shape[0]
    pad = jnp.full((npad_e,), PAD_IDX, jnp.int32)
    src_f = jnp.concatenate([src, pad])
    dst_f = jnp.concatenate([dst, pad])
    src_p = src_f.reshape(NW, KCH, CH)
    dst_p = dst_f.reshape(NW, KCH, CH)
    half0 = dst_f < SHALF
    src_m = jnp.stack([jnp.where(half0, src_f, IGN),
                       jnp.where(half0, IGN, src_f)])
    dst_m = jnp.stack([jnp.where(half0, dst_f, IGN),
                       jnp.where(half0, IGN, dst_f - SHALF)])
    src_m = src_m.reshape(2, NW, KCH, CH)
    dst_m = dst_m.reshape(2, NW, KCH, CH)

    x_p = jnp.pad(x, ((0, NPAD - N), (0, 0)))
    b2 = b_gc.reshape(1, D)
    zD = jnp.zeros((NPAD, D), jnp.float32)
    lanes = jnp.arange(D)
    olo = jnp.broadcast_to((lanes < 64).astype(jnp.float32), (CH, D))
    ohi = jnp.broadcast_to((lanes >= 64).astype(jnp.float32), (CH, D))

    spmm_kernel, deg_kernel = _sc_kernels()
    degs = deg_kernel(src_p, dst_p, zD, olo, ohi)

    kpre = pl.pallas_call(
        _kpre_body,
        grid=(GRID,),
        in_specs=[
            _row_spec(),
            pl.BlockSpec((D, D), lambda i: (0, 0)),
            _agg_spec(),
        ],
        out_specs=_row_spec(),
        out_shape=jax.ShapeDtypeStruct((NPAD, D), jnp.float32),
    )
    sup = kpre(x_p, W_gc, degs)

    kmid = pl.pallas_call(
        _kmid_body,
        grid=(GRID,),
        in_specs=[
            _row_spec(),
            _agg_spec(),
            _agg_spec(),
            pl.BlockSpec((D, D), lambda i: (0, 0)),
            pl.BlockSpec((1, D), lambda i: (0, 0)),
        ],
        out_specs=[_row_spec(), _row_spec()],
        out_shape=[jax.ShapeDtypeStruct((NPAD, D), jnp.float32)] * 2,
    )

    h = x_p
    for _ in range(NITE - 1):
        agg = spmm_kernel(sup, src_m, dst_m, zD)
        h, sup = kmid(h, agg, degs, W_gc, b2)

    agg = spmm_kernel(sup, src_m, dst_m, zD)

    kpost = pl.pallas_call(
        _kpost_body,
        grid=(GRID,),
        in_specs=[
            _row_spec(),
            _agg_spec(),
            _agg_spec(),
            pl.BlockSpec((1, D), lambda i: (0, 0)),
            pl.BlockSpec((D, NCLS), lambda i: (0, 0)),
        ],
        out_specs=pl.BlockSpec((RB, NCLS), lambda i: (i, 0)),
        out_shape=jax.ShapeDtypeStruct((NPAD, NCLS), jnp.float32),
    )
    out = kpost(h, agg, degs, b2, W_lin)
    return out[:N]


# speedup vs baseline: 2.2493x; 1.5188x over previous
import functools

import jax
import jax.numpy as jnp
from jax import lax
from jax.experimental import pallas as pl
from jax.experimental.pallas import tpu as pltpu
from jax.experimental.pallas import tpu_sc as plsc

N = 10000
D = 128
NCLS = 40
NITE = 4
SMOOTH = 0.5

NPAD = 10240
SHALF = NPAD // 2
PAD_IDX = N
NW = 32
CH = 128
KCH = 80
QN = 5
QCH = KCH // QN
NB = 4
EPAD = NW * KCH * CH
RB = 1280
GRID = NPAD // RB
SROWS = NPAD // 16
HROWS = SHALF // 16
IGN = -1


def _wid():
    return lax.axis_index("s") * 2 + lax.axis_index("c")


def _spmm_body(sup_hbm, src_hbm, dst_hbm, zD_hbm, agg_hbm,
               src_v, dst_v, buf0, buf1, sup_sh, agg_sh, sem0, sem1):
    cid = lax.axis_index("c")
    sid = lax.axis_index("s")
    wid = _wid()

    def _sref(row):
        return sup_sh.at[plsc.Indices(src_v.at[row], ignored_value=IGN)]

    def _dref(row):
        return agg_sh.at[plsc.Indices(dst_v.at[row], ignored_value=IGN)]

    for dh in range(2):
        pltpu.sync_copy(zD_hbm.at[pl.ds(sid * HROWS, HROWS)],
                        agg_sh.at[pl.ds(sid * HROWS, HROWS)])
        for sh in range(2):
            pltpu.sync_copy(
                sup_hbm.at[pl.ds(sh * SHALF + sid * HROWS, HROWS)],
                sup_sh.at[pl.ds(sid * HROWS, HROWS)])
            plsc.subcore_barrier()

            def quarter(q, _):
                pltpu.sync_copy(src_hbm.at[dh, sh, wid, pl.ds(q * QCH, QCH)],
                                src_v)
                pltpu.sync_copy(dst_hbm.at[dh, sh, wid, pl.ds(q * QCH, QCH)],
                                dst_v)
                pltpu.async_copy(_sref(0), buf0, sem0)

                def pair(j, _):
                    pltpu.async_copy(_sref(2 * j + 1), buf1, sem1)
                    pltpu.make_async_copy(_sref(2 * j), buf0, sem0).wait()
                    pltpu.sync_copy(buf0, _dref(2 * j), add=True)
                    pltpu.async_copy(_sref(2 * j + 2), buf0, sem0)
                    pltpu.make_async_copy(_sref(2 * j + 1), buf1, sem1).wait()
                    pltpu.sync_copy(buf1, _dref(2 * j + 1), add=True)
                    return 0
                lax.fori_loop(0, QCH // 2 - 1, pair, 0)

                pltpu.async_copy(_sref(QCH - 1), buf1, sem1)
                pltpu.make_async_copy(_sref(QCH - 2), buf0, sem0).wait()
                pltpu.sync_copy(buf0, _dref(QCH - 2), add=True)
                pltpu.make_async_copy(_sref(QCH - 1), buf1, sem1).wait()
                pltpu.sync_copy(buf1, _dref(QCH - 1), add=True)
                return 0
            lax.fori_loop(0, QN, quarter, 0)
            plsc.subcore_barrier()

        pltpu.sync_copy(
            agg_sh.at[pl.ds(sid * HROWS, HROWS)],
            agg_hbm.at[cid, pl.ds(dh * SHALF + sid * HROWS, HROWS)])
        plsc.subcore_barrier()


def _deg_body(src_hbm, dst_hbm, zD_hbm, olo_hbm, ohi_hbm, deg_hbm,
              src_v, dst_v, olo_v, ohi_v, acc_sh, sem2, sem3):
    cid = lax.axis_index("c")
    sid = lax.axis_index("s")
    wid = sid * 2 + cid

    pltpu.sync_copy(zD_hbm.at[pl.ds(sid * SROWS, SROWS)],
                    acc_sh.at[pl.ds(sid * SROWS, SROWS)])
    pltpu.sync_copy(olo_hbm, olo_v)
    pltpu.sync_copy(ohi_hbm, ohi_v)
    plsc.subcore_barrier()

    def quarter(q, _):
        pltpu.sync_copy(src_hbm.at[wid, pl.ds(q * QCH, QCH)], src_v)
        pltpu.sync_copy(dst_hbm.at[wid, pl.ds(q * QCH, QCH)], dst_v)

        def fire(j, _):
            pltpu.async_copy(olo_v, acc_sh.at[src_v.at[j]], sem2, add=True)
            pltpu.async_copy(ohi_v, acc_sh.at[dst_v.at[j]], sem3, add=True)
            return 0
        lax.fori_loop(0, QCH, fire, 0)

        def drain(j, _):
            pltpu.make_async_copy(olo_v, acc_sh.at[src_v.at[j]], sem2).wait()
            pltpu.make_async_copy(ohi_v, acc_sh.at[dst_v.at[j]], sem3).wait()
            return 0
        lax.fori_loop(0, QCH, drain, 0)
        return 0
    lax.fori_loop(0, QN, quarter, 0)
    plsc.subcore_barrier()

    pltpu.sync_copy(acc_sh.at[pl.ds(sid * SROWS, SROWS)],
                    deg_hbm.at[cid, pl.ds(sid * SROWS, SROWS)])


@functools.lru_cache(maxsize=1)
def _sc_kernels():
    mesh = plsc.VectorSubcoreMesh(core_axis_name="c", subcore_axis_name="s",
                                  num_cores=2, num_subcores=16)
    spmm = pl.kernel(
        _spmm_body,
        out_type=jax.ShapeDtypeStruct((2, NPAD, D), jnp.float32),
        mesh=mesh,
        scratch_types=[
            pltpu.VMEM((QCH, CH), jnp.int32),
            pltpu.VMEM((QCH, CH), jnp.int32),
            pltpu.VMEM((CH, D), jnp.float32),
            pltpu.VMEM((CH, D), jnp.float32),
            pltpu.VMEM_SHARED((SHALF, D), jnp.float32),
            pltpu.VMEM_SHARED((SHALF, D), jnp.float32),
            pltpu.SemaphoreType.DMA,
            pltpu.SemaphoreType.DMA,
        ],
    )
    deg = pl.kernel(
        _deg_body,
        out_type=jax.ShapeDtypeStruct((2, NPAD, D), jnp.float32),
        mesh=mesh,
        scratch_types=[
            pltpu.VMEM((QCH, CH), jnp.int32),
            pltpu.VMEM((QCH, CH), jnp.int32),
            pltpu.VMEM((CH, D), jnp.float32),
            pltpu.VMEM((CH, D), jnp.float32),
            pltpu.VMEM_SHARED((NPAD, D), jnp.float32),
            pltpu.SemaphoreType.DMA,
            pltpu.SemaphoreType.DMA,
        ],
    )
    return spmm, deg


def _col(dref, lane):
    s = dref[0, :, lane:lane + 1] + dref[1, :, lane:lane + 1]
    return lax.rsqrt(jnp.maximum(s, 1.0))


def _kpre_body(x_ref, w_ref, deg_ref, sup_ref):
    f = _col(deg_ref, 0)
    sup_ref[...] = jnp.dot(x_ref[...], w_ref[...],
                           preferred_element_type=jnp.float32) * f


def _kmid_body(h_ref, agg_ref, deg_ref, w_ref, b_ref,
               h_out, sup_out):
    f = _col(deg_ref, 0)
    g = _col(deg_ref, 64)
    agg = agg_ref[0] + agg_ref[1]
    aggn = g * agg + b_ref[...]
    hn = jnp.maximum(SMOOTH * h_ref[...] + (1.0 - SMOOTH) * aggn, 0.0)
    h_out[...] = hn
    sup_out[...] = jnp.dot(hn, w_ref[...],
                           preferred_element_type=jnp.float32) * f


def _kpost_body(h_ref, agg_ref, deg_ref, b_ref, wl_ref, out_ref):
    g = _col(deg_ref, 64)
    agg = agg_ref[0] + agg_ref[1]
    aggn = g * agg + b_ref[...]
    hn = jnp.maximum(SMOOTH * h_ref[...] + (1.0 - SMOOTH) * aggn, 0.0)
    logits = jnp.dot(hn, wl_ref[...], preferred_element_type=jnp.float32)
    m = jnp.max(logits, axis=1, keepdims=True)
    lse = m + jnp.log(jnp.sum(jnp.exp(logits - m), axis=1, keepdims=True))
    out_ref[...] = logits - lse


def _row_spec():
    return pl.BlockSpec((RB, D), lambda i: (i, 0))


def _agg_spec():
    return pl.BlockSpec((2, RB, D), lambda i: (0, i, 0))


def kernel(x, edge_index, W_gc, b_gc, W_lin):
    src = edge_index[0]
    dst = edge_index[1]
    npad_e = EPAD - src.shape[0]
    pad = jnp.full((npad_e,), PAD_IDX, jnp.int32)
    src_f = jnp.concatenate([src, pad])
    dst_f = jnp.concatenate([dst, pad])
    src_p = src_f.reshape(NW, KCH, CH)
    dst_p = dst_f.reshape(NW, KCH, CH)
    quads = []
    for dh in range(2):
        row = []
        for sh in range(2):
            act = jnp.logical_and((dst_f < SHALF) == (dh == 0),
                                  (src_f < SHALF) == (sh == 0))
            row.append((jnp.where(act, src_f - sh * SHALF, IGN),
                        jnp.where(act, dst_f - dh * SHALF, IGN)))
        quads.append(row)
    src_m = jnp.stack([jnp.stack([q[0] for q in row]) for row in quads])
    dst_m = jnp.stack([jnp.stack([q[1] for q in row]) for row in quads])
    src_m = src_m.reshape(2, 2, NW, KCH, CH)
    dst_m = dst_m.reshape(2, 2, NW, KCH, CH)

    x_p = jnp.pad(x, ((0, NPAD - N), (0, 0)))
    b2 = b_gc.reshape(1, D)
    zD = jnp.zeros((NPAD, D), jnp.float32)
    lanes = jnp.arange(D)
    olo = jnp.broadcast_to((lanes < 64).astype(jnp.float32), (CH, D))
    ohi = jnp.broadcast_to((lanes >= 64).astype(jnp.float32), (CH, D))

    spmm_kernel, deg_kernel = _sc_kernels()
    degs = deg_kernel(src_p, dst_p, zD, olo, ohi)

    kpre = pl.pallas_call(
        _kpre_body,
        grid=(GRID,),
        in_specs=[
            _row_spec(),
            pl.BlockSpec((D, D), lambda i: (0, 0)),
            _agg_spec(),
        ],
        out_specs=_row_spec(),
        out_shape=jax.ShapeDtypeStruct((NPAD, D), jnp.float32),
    )
    sup = kpre(x_p, W_gc, degs)

    kmid = pl.pallas_call(
        _kmid_body,
        grid=(GRID,),
        in_specs=[
            _row_spec(),
            _agg_spec(),
            _agg_spec(),
            pl.BlockSpec((D, D), lambda i: (0, 0)),
            pl.BlockSpec((1, D), lambda i: (0, 0)),
        ],
        out_specs=[_row_spec(), _row_spec()],
        out_shape=[jax.ShapeDtypeStruct((NPAD, D), jnp.float32)] * 2,
    )

    h = x_p
    for _ in range(NITE - 1):
        agg = spmm_kernel(sup, src_m, dst_m, zD)
        h, sup = kmid(h, agg, degs, W_gc, b2)

    agg = spmm_kernel(sup, src_m, dst_m, zD)

    kpost = pl.pallas_call(
        _kpost_body,
        grid=(GRID,),
        in_specs=[
            _row_spec(),
            _agg_spec(),
            _agg_spec(),
            pl.BlockSpec((1, D), lambda i: (0, 0)),
            pl.BlockSpec((D, NCLS), lambda i: (0, 0)),
        ],
        out_specs=pl.BlockSpec((RB, NCLS), lambda i: (i, 0)),
        out_shape=jax.ShapeDtypeStruct((NPAD, NCLS), jnp.float32),
    )
    out = kpost(h, agg, degs, b2, W_lin)
    return out[:N]


# speedup vs baseline: 2.4209x; 1.0763x over previous
import functools

import jax
import jax.numpy as jnp
from jax import lax
from jax.experimental import pallas as pl
from jax.experimental.pallas import tpu as pltpu
from jax.experimental.pallas import tpu_sc as plsc

N = 10000
D = 128
NCLS = 40
NITE = 4
SMOOTH = 0.5

NPAD = 10240
SHALF = NPAD // 2
PAD_IDX = N
NW = 32
CH = 128
KCH = 80
QN = 2
QCH = KCH // QN
NB = 4
EPAD = NW * KCH * CH
RB = 1280
GRID = NPAD // RB
SROWS = NPAD // 16
HROWS = SHALF // 16
IGN = -1


def _wid():
    return lax.axis_index("s") * 2 + lax.axis_index("c")


def _spmm_body(sup_hbm, src_hbm, dst_hbm, zD_hbm, agg_hbm,
               src_v, dst_v, buf0, buf1, sup_sh, agg_sh, sem0, sem1):
    cid = lax.axis_index("c")
    sid = lax.axis_index("s")
    wid = _wid()

    def _sref(row):
        return sup_sh.at[plsc.Indices(src_v.at[row], ignored_value=IGN)]

    def _dref(row):
        return agg_sh.at[plsc.Indices(dst_v.at[row], ignored_value=IGN)]

    for dh in range(2):
        pltpu.sync_copy(zD_hbm.at[pl.ds(sid * HROWS, HROWS)],
                        agg_sh.at[pl.ds(sid * HROWS, HROWS)])
        for sh in range(2):
            pltpu.sync_copy(
                sup_hbm.at[pl.ds(sh * SHALF + sid * HROWS, HROWS)],
                sup_sh.at[pl.ds(sid * HROWS, HROWS)])
            plsc.subcore_barrier()

            def quarter(q, _):
                pltpu.sync_copy(src_hbm.at[dh, sh, wid, pl.ds(q * QCH, QCH)],
                                src_v)
                pltpu.sync_copy(dst_hbm.at[dh, sh, wid, pl.ds(q * QCH, QCH)],
                                dst_v)
                pltpu.async_copy(_sref(0), buf0, sem0)

                def pair(j, _):
                    pltpu.async_copy(_sref(2 * j + 1), buf1, sem1)
                    pltpu.make_async_copy(_sref(2 * j), buf0, sem0).wait()
                    pltpu.sync_copy(buf0, _dref(2 * j), add=True)
                    pltpu.async_copy(_sref(2 * j + 2), buf0, sem0)
                    pltpu.make_async_copy(_sref(2 * j + 1), buf1, sem1).wait()
                    pltpu.sync_copy(buf1, _dref(2 * j + 1), add=True)
                    return 0
                lax.fori_loop(0, QCH // 2 - 1, pair, 0)

                pltpu.async_copy(_sref(QCH - 1), buf1, sem1)
                pltpu.make_async_copy(_sref(QCH - 2), buf0, sem0).wait()
                pltpu.sync_copy(buf0, _dref(QCH - 2), add=True)
                pltpu.make_async_copy(_sref(QCH - 1), buf1, sem1).wait()
                pltpu.sync_copy(buf1, _dref(QCH - 1), add=True)
                return 0
            lax.fori_loop(0, QN, quarter, 0)
            plsc.subcore_barrier()

        pltpu.sync_copy(
            agg_sh.at[pl.ds(sid * HROWS, HROWS)],
            agg_hbm.at[cid, pl.ds(dh * SHALF + sid * HROWS, HROWS)])
        plsc.subcore_barrier()


def _deg_body(src_hbm, dst_hbm, zD_hbm, olo_hbm, ohi_hbm, deg_hbm,
              src_v, dst_v, olo_v, ohi_v, acc_sh, sem2, sem3):
    cid = lax.axis_index("c")
    sid = lax.axis_index("s")
    wid = sid * 2 + cid

    pltpu.sync_copy(zD_hbm.at[pl.ds(sid * SROWS, SROWS)],
                    acc_sh.at[pl.ds(sid * SROWS, SROWS)])
    pltpu.sync_copy(olo_hbm, olo_v)
    pltpu.sync_copy(ohi_hbm, ohi_v)
    plsc.subcore_barrier()

    def quarter(q, _):
        pltpu.sync_copy(src_hbm.at[wid, pl.ds(q * QCH, QCH)], src_v)
        pltpu.sync_copy(dst_hbm.at[wid, pl.ds(q * QCH, QCH)], dst_v)

        def fire(j, _):
            pltpu.async_copy(olo_v, acc_sh.at[src_v.at[j]], sem2, add=True)
            pltpu.async_copy(ohi_v, acc_sh.at[dst_v.at[j]], sem3, add=True)
            return 0
        lax.fori_loop(0, QCH, fire, 0)

        def drain(j, _):
            pltpu.make_async_copy(olo_v, acc_sh.at[src_v.at[j]], sem2).wait()
            pltpu.make_async_copy(ohi_v, acc_sh.at[dst_v.at[j]], sem3).wait()
            return 0
        lax.fori_loop(0, QCH, drain, 0)
        return 0
    lax.fori_loop(0, QN, quarter, 0)
    plsc.subcore_barrier()

    pltpu.sync_copy(acc_sh.at[pl.ds(sid * SROWS, SROWS)],
                    deg_hbm.at[cid, pl.ds(sid * SROWS, SROWS)])


@functools.lru_cache(maxsize=1)
def _sc_kernels():
    mesh = plsc.VectorSubcoreMesh(core_axis_name="c", subcore_axis_name="s",
                                  num_cores=2, num_subcores=16)
    spmm = pl.kernel(
        _spmm_body,
        out_type=jax.ShapeDtypeStruct((2, NPAD, D), jnp.float32),
        mesh=mesh,
        scratch_types=[
            pltpu.VMEM((QCH, CH), jnp.int32),
            pltpu.VMEM((QCH, CH), jnp.int32),
            pltpu.VMEM((CH, D), jnp.float32),
            pltpu.VMEM((CH, D), jnp.float32),
            pltpu.VMEM_SHARED((SHALF, D), jnp.float32),
            pltpu.VMEM_SHARED((SHALF, D), jnp.float32),
            pltpu.SemaphoreType.DMA,
            pltpu.SemaphoreType.DMA,
        ],
    )
    deg = pl.kernel(
        _deg_body,
        out_type=jax.ShapeDtypeStruct((2, NPAD, D), jnp.float32),
        mesh=mesh,
        scratch_types=[
            pltpu.VMEM((QCH, CH), jnp.int32),
            pltpu.VMEM((QCH, CH), jnp.int32),
            pltpu.VMEM((CH, D), jnp.float32),
            pltpu.VMEM((CH, D), jnp.float32),
            pltpu.VMEM_SHARED((NPAD, D), jnp.float32),
            pltpu.SemaphoreType.DMA,
            pltpu.SemaphoreType.DMA,
        ],
    )
    return spmm, deg


def _col(dref, lane):
    s = dref[0, :, lane:lane + 1] + dref[1, :, lane:lane + 1]
    return lax.rsqrt(jnp.maximum(s, 1.0))


def _kpre_body(x_ref, w_ref, deg_ref, sup_ref):
    f = _col(deg_ref, 0)
    sup_ref[...] = jnp.dot(x_ref[...], w_ref[...],
                           preferred_element_type=jnp.float32) * f


def _kmid_body(h_ref, agg_ref, deg_ref, w_ref, b_ref,
               h_out, sup_out):
    f = _col(deg_ref, 0)
    g = _col(deg_ref, 64)
    agg = agg_ref[0] + agg_ref[1]
    aggn = g * agg + b_ref[...]
    hn = jnp.maximum(SMOOTH * h_ref[...] + (1.0 - SMOOTH) * aggn, 0.0)
    h_out[...] = hn
    sup_out[...] = jnp.dot(hn, w_ref[...],
                           preferred_element_type=jnp.float32) * f


def _kpost_body(h_ref, agg_ref, deg_ref, b_ref, wl_ref, out_ref):
    g = _col(deg_ref, 64)
    agg = agg_ref[0] + agg_ref[1]
    aggn = g * agg + b_ref[...]
    hn = jnp.maximum(SMOOTH * h_ref[...] + (1.0 - SMOOTH) * aggn, 0.0)
    logits = jnp.dot(hn, wl_ref[...], preferred_element_type=jnp.float32)
    m = jnp.max(logits, axis=1, keepdims=True)
    lse = m + jnp.log(jnp.sum(jnp.exp(logits - m), axis=1, keepdims=True))
    out_ref[...] = logits - lse


def _row_spec():
    return pl.BlockSpec((RB, D), lambda i: (i, 0))


def _agg_spec():
    return pl.BlockSpec((2, RB, D), lambda i: (0, i, 0))


def kernel(x, edge_index, W_gc, b_gc, W_lin):
    src = edge_index[0]
    dst = edge_index[1]
    npad_e = EPAD - src.shape[0]
    pad = jnp.full((npad_e,), PAD_IDX, jnp.int32)
    src_f = jnp.concatenate([src, pad])
    dst_f = jnp.concatenate([dst, pad])
    src_p = src_f.reshape(NW, KCH, CH)
    dst_p = dst_f.reshape(NW, KCH, CH)
    quads = []
    for dh in range(2):
        row = []
        for sh in range(2):
            act = jnp.logical_and((dst_f < SHALF) == (dh == 0),
                                  (src_f < SHALF) == (sh == 0))
            row.append((jnp.where(act, src_f - sh * SHALF, IGN),
                        jnp.where(act, dst_f - dh * SHALF, IGN)))
        quads.append(row)
    src_m = jnp.stack([jnp.stack([q[0] for q in row]) for row in quads])
    dst_m = jnp.stack([jnp.stack([q[1] for q in row]) for row in quads])
    src_m = src_m.reshape(2, 2, NW, KCH, CH)
    dst_m = dst_m.reshape(2, 2, NW, KCH, CH)

    x_p = jnp.pad(x, ((0, NPAD - N), (0, 0)))
    b2 = b_gc.reshape(1, D)
    zD = jnp.zeros((NPAD, D), jnp.float32)
    lanes = jnp.arange(D)
    olo = jnp.broadcast_to((lanes < 64).astype(jnp.float32), (CH, D))
    ohi = jnp.broadcast_to((lanes >= 64).astype(jnp.float32), (CH, D))

    spmm_kernel, deg_kernel = _sc_kernels()
    degs = deg_kernel(src_p, dst_p, zD, olo, ohi)

    kpre = pl.pallas_call(
        _kpre_body,
        grid=(GRID,),
        in_specs=[
            _row_spec(),
            pl.BlockSpec((D, D), lambda i: (0, 0)),
            _agg_spec(),
        ],
        out_specs=_row_spec(),
        out_shape=jax.ShapeDtypeStruct((NPAD, D), jnp.float32),
    )
    sup = kpre(x_p, W_gc, degs)

    kmid = pl.pallas_call(
        _kmid_body,
        grid=(GRID,),
        in_specs=[
            _row_spec(),
            _agg_spec(),
            _agg_spec(),
            pl.BlockSpec((D, D), lambda i: (0, 0)),
            pl.BlockSpec((1, D), lambda i: (0, 0)),
        ],
        out_specs=[_row_spec(), _row_spec()],
        out_shape=[jax.ShapeDtypeStruct((NPAD, D), jnp.float32)] * 2,
    )

    h = x_p
    for _ in range(NITE - 1):
        agg = spmm_kernel(sup, src_m, dst_m, zD)
        h, sup = kmid(h, agg, degs, W_gc, b2)

    agg = spmm_kernel(sup, src_m, dst_m, zD)

    kpost = pl.pallas_call(
        _kpost_body,
        grid=(GRID,),
        in_specs=[
            _row_spec(),
            _agg_spec(),
            _agg_spec(),
            pl.BlockSpec((1, D), lambda i: (0, 0)),
            pl.BlockSpec((D, NCLS), lambda i: (0, 0)),
        ],
        out_specs=pl.BlockSpec((RB, NCLS), lambda i: (i, 0)),
        out_shape=jax.ShapeDtypeStruct((NPAD, NCLS), jnp.float32),
    )
    out = kpost(h, agg, degs, b2, W_lin)
    return out[:N]
